# baseline (speedup 1.0000x reference)
import functools

import jax
import jax.numpy as jnp
from jax import lax
from jax.experimental import pallas as pl
from jax.experimental.pallas import tpu as pltpu
from jax.experimental.pallas import tpu_sc as plsc

_NULL_ROW = 1000000
_HIDDEN = 64
_BATCH = 16384
_LANES = 16
_CHUNK = 128

_info = plsc.get_sparse_core_info()
_NC, _NS = _info.num_cores, _info.num_subcores
_NW = _NC * _NS
_ROWS = _BATCH // _CHUNK
_RPW = _ROWS // _NW

_mesh = plsc.VectorSubcoreMesh(core_axis_name="c", subcore_axis_name="s")


@functools.partial(
    pl.kernel,
    mesh=_mesh,
    out_type=jax.ShapeDtypeStruct((_ROWS, _CHUNK, _HIDDEN), jnp.float32),
    scratch_types=[
        pltpu.VMEM((_RPW, _CHUNK), jnp.int32),
        pltpu.VMEM((_RPW, _CHUNK), jnp.int32),
        pltpu.VMEM((_RPW, _CHUNK, _HIDDEN), jnp.float32),
        pltpu.SemaphoreType.DMA,
    ],
    compiler_params=pltpu.CompilerParams(use_tc_tiling_on_sc=False),
)
def _lookup(lbl_hbm, fd_hbm, table_hbm, out_hbm, lbl_v, fd_v, rows_v, sem):
    wid = lax.axis_index("s") * _NC + lax.axis_index("c")
    base = wid * _RPW
    pltpu.sync_copy(lbl_hbm.at[pl.ds(base, _RPW)], lbl_v)
    pltpu.sync_copy(fd_hbm.at[pl.ds(base, _RPW)], fd_v)
    for j in range(_RPW):
        for i in range(_CHUNK // _LANES):
            sl = (j, pl.ds(i * _LANES, _LANES))
            lbl_v[sl] = jnp.where(fd_v[sl] == 1, _NULL_ROW, lbl_v[sl])
    copies = [
        pltpu.async_copy(table_hbm.at[lbl_v.at[j]], rows_v.at[j], sem)
        for j in range(_RPW)
    ]
    for c in copies:
        c.wait()
    pltpu.sync_copy(rows_v, out_hbm.at[pl.ds(base, _RPW)])


def kernel(labels, train, force_drop_ids, embedding_table):
    del train
    lbl2 = labels.reshape(_ROWS, _CHUNK).astype(jnp.int32)
    fd2 = force_drop_ids.reshape(_ROWS, _CHUNK).astype(jnp.int32)
    out = _lookup(lbl2, fd2, embedding_table)
    return out.reshape(_BATCH, _HIDDEN)


# speedup vs baseline: 1.1511x; 1.1511x over previous
import functools

import jax
import jax.numpy as jnp
from jax import lax
from jax.experimental import pallas as pl
from jax.experimental.pallas import tpu as pltpu
from jax.experimental.pallas import tpu_sc as plsc

_NULL_ROW = 1000000
_HIDDEN = 64
_BATCH = 16384

_info = plsc.get_sparse_core_info()
_NC, _NS = _info.num_cores, _info.num_subcores
_NW = _NC * _NS
_BPW = _BATCH // _NW

_mesh = plsc.VectorSubcoreMesh(core_axis_name="c", subcore_axis_name="s")


@functools.partial(
    pl.kernel,
    mesh=_mesh,
    out_type=jax.ShapeDtypeStruct((_NW, _BPW, _HIDDEN), jnp.float32),
    scratch_types=[
        pltpu.VMEM((_BPW,), jnp.int32),
        pltpu.VMEM((_BPW,), jnp.int32),
        pltpu.VMEM((_BPW, _HIDDEN), jnp.float32),
        pltpu.SemaphoreType.DMA,
    ],
)
def _lookup(lbl_hbm, fd_hbm, table_hbm, out_hbm, lbl_v, fd_v, rows_v, sem):
    wid = lax.axis_index("s") * _NC + lax.axis_index("c")
    pltpu.sync_copy(lbl_hbm.at[wid], lbl_v)
    pltpu.sync_copy(fd_hbm.at[wid], fd_v)
    for i in range(_BPW // 16):
        sl = pl.ds(i * 16, 16)
        lbl_v[sl] = jnp.where(fd_v[sl] == 1, _NULL_ROW, lbl_v[sl])

    def issue(g, carry):
        vec = lbl_v[pl.ds(g * 16, 16)]
        for j in range(16):
            pltpu.async_copy(table_hbm.at[vec[j]], rows_v.at[g * 16 + j], sem)
        return carry

    lax.fori_loop(0, _BPW // 16, issue, 0)
    pltpu.make_async_copy(table_hbm.at[pl.ds(0, _BPW)], rows_v, sem).wait()
    pltpu.sync_copy(rows_v, out_hbm.at[wid])


def kernel(labels, train, force_drop_ids, embedding_table):
    del train
    lbl2 = labels.reshape(_NW, _BPW).astype(jnp.int32)
    fd2 = force_drop_ids.reshape(_NW, _BPW).astype(jnp.int32)
    out = _lookup(lbl2, fd2, embedding_table)
    return out.reshape(_BATCH, _HIDDEN)
